# Initial kernel scaffold; baseline (speedup 1.0000x reference)
import jax
import jax.numpy as jnp
from jax.experimental import pallas as pl


def kernel(input_ids, weight, eoi_embedding):
    raise NotImplementedError("write your pallas kernel here")


# speedup vs baseline: 1.0742x; 1.0742x over previous
import jax
import jax.numpy as jnp
from jax import lax
from jax.experimental import pallas as pl
from jax.experimental.pallas import tpu as pltpu
from jax.experimental.pallas import tpu_sc as plsc

D = 128
EMBED_SCALE = 11.313708498984761
EOI = 256000
NC, NS, L = 2, 16, 16
NW = NC * NS
CHUNK = 128
NBUF = 3
CG = D // L


def _sc_body(idx_hbm, idx3_hbm, w_hbm, eoi_hbm, out_hbm, idx_v, idx3_v, eoi_v, *bufs_sems):
    cnt_s = bufs_sems[-1]
    bufs_sems = bufs_sems[:-1]
    nch = idx_hbm.shape[1]
    bufs = bufs_sems[:NBUF]
    gsems = bufs_sems[NBUF:2 * NBUF]
    osems = bufs_sems[2 * NBUF:3 * NBUF]
    per_w = nch * CHUNK
    wid = lax.axis_index("s") * NC + lax.axis_index("c")
    base = wid * per_w

    pltpu.sync_copy(idx_hbm.at[wid], idx_v)
    pltpu.sync_copy(idx3_hbm.at[wid], idx3_v)
    pltpu.sync_copy(eoi_hbm, eoi_v)
    ev = [eoi_v[pl.ds(c * L, L)] for c in range(CG)]

    def gather(ch, b):
        return pltpu.make_async_copy(w_hbm.at[idx_v.at[ch]], bufs[b], gsems[b])

    def out_copy(ch, b):
        dst = out_hbm.at[pl.ds(base + ch * CHUNK, CHUNK)]
        return pltpu.make_async_copy(bufs[b], dst, osems[b])

    for b in range(min(NBUF, nch)):
        gather(b, b).start()

    for ch in range(nch):
        b = ch % NBUF
        gather(ch, b).wait()
        buf = bufs[b]

        def grp_body(g, carry):
            iv = idx3_v[ch, g, :]
            fm = (1 - jnp.minimum(jnp.abs(iv - EOI), 1)).astype(jnp.float32)
            for r0 in range(L):
                fs = fm[r0]
                sc = EMBED_SCALE * (1.0 - fs)
                row = g * L + r0
                for c in range(CG):
                    buf[row, pl.ds(c * L, L)] = (
                        buf[row, pl.ds(c * L, L)] * sc + ev[c] * fs)
            return carry

        lax.fori_loop(0, CHUNK // L, grp_body, jnp.int32(0))
        out_copy(ch, b).start()
        nxt = ch + NBUF
        if nxt < nch:
            out_copy(ch, b).wait()
            gather(nxt, b).start()

    for ch in range(max(0, nch - NBUF), nch):
        out_copy(ch, ch % NBUF).wait()


def kernel(input_ids, weight, eoi_embedding):
    batch, seq = input_ids.shape
    tot = batch * seq
    nch = tot // (NW * CHUNK)
    idx = input_ids.reshape(NW, nch, CHUNK).astype(jnp.int32)
    mesh = plsc.VectorSubcoreMesh(core_axis_name="c", subcore_axis_name="s")
    out = pl.kernel(
        _sc_body,
        out_type=jax.ShapeDtypeStruct((tot, D), jnp.float32),
        mesh=mesh,
        scratch_types=(
            [pltpu.VMEM((nch, CHUNK), jnp.int32),
             pltpu.VMEM((nch, CHUNK // L, L), jnp.int32),
             pltpu.VMEM((D,), jnp.float32)]
            + [pltpu.VMEM((CHUNK, D), jnp.float32)] * NBUF
            + [pltpu.SemaphoreType.DMA] * (2 * NBUF)
            + [pltpu.SMEM((1,), jnp.int32)]
                    ),
    )(idx, idx.reshape(NW, nch, CHUNK // L, L), weight, eoi_embedding.astype(jnp.float32))
    return out.reshape(batch, seq, D)


# speedup vs baseline: 1.1001x; 1.0241x over previous
import jax
import jax.numpy as jnp
from jax import lax
from jax.experimental import pallas as pl
from jax.experimental.pallas import tpu as pltpu
from jax.experimental.pallas import tpu_sc as plsc

D = 128
EMBED_SCALE = 11.313708498984761
EOI = 256000
NC, NS, L = 2, 16, 16
NW = NC * NS
CHUNK = 128
NBUF = 3
CG = D // L


def _sc_body(idx_hbm, idx3_hbm, w_hbm, eoi_hbm, out_hbm, idx_v, idx3_v,
             eoi_v, *bufs_sems):
    nch = idx_hbm.shape[1]
    ibufs = bufs_sems[:NBUF]
    obufs = bufs_sems[NBUF:2 * NBUF]
    gsems = bufs_sems[2 * NBUF:3 * NBUF]
    osems = bufs_sems[3 * NBUF:4 * NBUF]
    per_w = nch * CHUNK
    wid = lax.axis_index("s") * NC + lax.axis_index("c")
    base = wid * per_w

    pltpu.sync_copy(idx_hbm.at[wid], idx_v)
    pltpu.sync_copy(idx3_hbm.at[wid], idx3_v)
    pltpu.sync_copy(eoi_hbm, eoi_v)
    ev = [eoi_v[pl.ds(c * L, L)] for c in range(CG)]

    def gather(ch, b):
        return pltpu.make_async_copy(w_hbm.at[idx_v.at[ch]], ibufs[b], gsems[b])

    def out_copy(ch, b):
        dst = out_hbm.at[pl.ds(base + ch * CHUNK, CHUNK)]
        return pltpu.make_async_copy(obufs[b], dst, osems[b])

    for b in range(min(NBUF, nch)):
        gather(b, b).start()

    for ch in range(nch):
        b = ch % NBUF
        gather(ch, b).wait()
        if ch - NBUF >= 0:
            out_copy(ch - NBUF, b).wait()
        bi = ibufs[b]
        bo = obufs[b]

        def grp_body(g, carry):
            iv = idx3_v[ch, g, :]
            fm = (1 - jnp.minimum(jnp.abs(iv - EOI), 1)).astype(jnp.float32)
            for r0 in range(L):
                fs = fm[r0]
                sc = EMBED_SCALE * (1.0 - fs)
                row = g * L + r0
                for c in range(CG):
                    bo[row, pl.ds(c * L, L)] = (
                        bi[row, pl.ds(c * L, L)] * sc + ev[c] * fs)
            return carry

        lax.fori_loop(0, CHUNK // L, grp_body, jnp.int32(0))
        out_copy(ch, b).start()
        nxt = ch + NBUF
        if nxt < nch:
            gather(nxt, b).start()

    for ch in range(max(0, nch - NBUF), nch):
        out_copy(ch, ch % NBUF).wait()


def kernel(input_ids, weight, eoi_embedding):
    batch, seq = input_ids.shape
    tot = batch * seq
    nch = tot // (NW * CHUNK)
    idx = input_ids.reshape(NW, nch, CHUNK).astype(jnp.int32)
    mesh = plsc.VectorSubcoreMesh(core_axis_name="c", subcore_axis_name="s")
    out = pl.kernel(
        _sc_body,
        out_type=jax.ShapeDtypeStruct((tot, D), jnp.float32),
        mesh=mesh,
        scratch_types=(
            [pltpu.VMEM((nch, CHUNK), jnp.int32),
             pltpu.VMEM((nch, CHUNK // L, L), jnp.int32),
             pltpu.VMEM((D,), jnp.float32)]
            + [pltpu.VMEM((CHUNK, D), jnp.float32)] * (2 * NBUF)
            + [pltpu.SemaphoreType.DMA] * (2 * NBUF)
        ),
    )(idx, idx.reshape(NW, nch, CHUNK // L, L), weight,
      eoi_embedding.astype(jnp.float32))
    return out.reshape(batch, seq, D)


# speedup vs baseline: 1.1224x; 1.0203x over previous
import jax
import jax.numpy as jnp
from jax import lax
from jax.experimental import pallas as pl
from jax.experimental.pallas import tpu as pltpu
from jax.experimental.pallas import tpu_sc as plsc

D = 128
EMBED_SCALE = 11.313708498984761
EOI = 256000
NC, NS, L = 2, 16, 16
NW = NC * NS
CHUNK = 128
NBUF = 3
CG = D // L


def _sc_body(idx_hbm, w_hbm, eoi_hbm, out_hbm, idx_v, eoi_v, *bufs_sems):
    nch = idx_hbm.shape[1]
    ibufs = bufs_sems[:NBUF]
    obufs = bufs_sems[NBUF:2 * NBUF]
    gsems = bufs_sems[2 * NBUF:3 * NBUF]
    osems = bufs_sems[3 * NBUF:4 * NBUF]
    per_w = nch * CHUNK
    wid = lax.axis_index("s") * NC + lax.axis_index("c")
    base = wid * per_w

    pltpu.sync_copy(idx_hbm.at[wid], idx_v)
    pltpu.sync_copy(eoi_hbm, eoi_v)
    ev = [eoi_v[pl.ds(c * L, L)] for c in range(CG)]

    def gather(ch, b):
        return pltpu.make_async_copy(w_hbm.at[idx_v.at[ch]], ibufs[b], gsems[b])

    seq = out_hbm.shape[1]
    b0 = base // seq
    soff = base % seq

    def out_copy(ch, b):
        dst = out_hbm.at[b0, pl.ds(soff + ch * CHUNK, CHUNK)]
        return pltpu.make_async_copy(obufs[b], dst, osems[b])

    for b in range(min(NBUF, nch)):
        gather(b, b).start()

    for ch in range(nch):
        b = ch % NBUF
        gather(ch, b).wait()
        if ch - NBUF >= 0:
            out_copy(ch - NBUF, b).wait()
        bi = ibufs[b]
        bo = obufs[b]

        def grp_body(g, carry):
            iv = idx_v[ch, pl.ds(g * L, L)]
            fm = (1 - jnp.minimum(jnp.abs(iv - EOI), 1)).astype(jnp.float32)
            for r0 in range(L):
                fs = fm[r0]
                sc = EMBED_SCALE * (1.0 - fs)
                row = g * L + r0
                for c in range(CG):
                    bo[row, pl.ds(c * L, L)] = (
                        bi[row, pl.ds(c * L, L)] * sc + ev[c] * fs)
            return carry

        lax.fori_loop(0, CHUNK // L, grp_body, jnp.int32(0))
        out_copy(ch, b).start()
        nxt = ch + NBUF
        if nxt < nch:
            gather(nxt, b).start()

    for ch in range(max(0, nch - NBUF), nch):
        out_copy(ch, ch % NBUF).wait()


def kernel(input_ids, weight, eoi_embedding):
    batch, seq = input_ids.shape
    tot = batch * seq
    nch = tot // (NW * CHUNK)
    idx = input_ids.reshape(NW, nch, CHUNK).astype(jnp.int32)
    mesh = plsc.VectorSubcoreMesh(core_axis_name="c", subcore_axis_name="s")
    out = pl.kernel(
        _sc_body,
        out_type=jax.ShapeDtypeStruct((batch, seq, D), jnp.float32),
        mesh=mesh,
        scratch_types=(
            [pltpu.VMEM((nch, CHUNK), jnp.int32),
             pltpu.VMEM((D,), jnp.float32)]
            + [pltpu.VMEM((CHUNK, D), jnp.float32)] * (2 * NBUF)
            + [pltpu.SemaphoreType.DMA] * (2 * NBUF)
        ),
    )(idx, weight, eoi_embedding.astype(jnp.float32))
    return out
